# Initial kernel scaffold; baseline (speedup 1.0000x reference)
import jax
import jax.numpy as jnp
from jax.experimental import pallas as pl


def kernel(x, batch_index, W1, b1, W2, b2, W3, b3):
    raise NotImplementedError("write your pallas kernel here")


# speedup vs baseline: 1.4564x; 1.4564x over previous
import functools

import jax
import jax.numpy as jnp
from jax import lax
from jax.experimental import pallas as pl
from jax.experimental.pallas import tpu as pltpu
from jax.experimental.pallas import tpu_sc as plsc

N = 100000
D = 512
S = 256
H1, H2 = 256, 128

T = 2000

RW = 6272
CH = 128
CW = 256


def _mlp_body(x_ref, seg_ref, w1_ref, b1_ref, w2_ref, b2_ref, w3_ref, b3_ref,
              y_ref, den_ref):
    i = pl.program_id(0)
    x = x_ref[...]
    h = jnp.maximum(
        jnp.dot(x, w1_ref[...], preferred_element_type=jnp.float32)
        + b1_ref[...], 0.0)
    h = jnp.maximum(
        jnp.dot(h, w2_ref[...], preferred_element_type=jnp.float32)
        + b2_ref[...], 0.0)
    logit = jnp.sum(h * w3_ref[...], axis=1, keepdims=True) + b3_ref[...]
    a = jnp.exp(logit)
    y_ref[...] = x * a

    seg = seg_ref[0, 0, :]
    onehot = (seg[None, :] == lax.broadcasted_iota(jnp.int32, (S, T), 0)
              ).astype(jnp.float32)
    a_b = jnp.broadcast_to(a, (a.shape[0], 128))

    @pl.when(i == 0)
    def _init():
        den_ref[...] = jnp.zeros_like(den_ref)

    den_ref[...] += jnp.dot(onehot, a_b, preferred_element_type=jnp.float32)


def _mlp_call(x, seg, W1, b1, W2, b2, W3, b3):
    grid = (N // T,)
    return pl.pallas_call(
        _mlp_body,
        grid=grid,
        in_specs=[
            pl.BlockSpec((T, D), lambda i: (i, 0)),
            pl.BlockSpec((1, 1, T), lambda i: (i, 0, 0)),
            pl.BlockSpec((D, H1), lambda i: (0, 0)),
            pl.BlockSpec((1, H1), lambda i: (0, 0)),
            pl.BlockSpec((H1, H2), lambda i: (0, 0)),
            pl.BlockSpec((1, H2), lambda i: (0, 0)),
            pl.BlockSpec((1, H2), lambda i: (0, 0)),
            pl.BlockSpec((1, 1), lambda i: (0, 0)),
        ],
        out_specs=[
            pl.BlockSpec((T, D), lambda i: (i, 0)),
            pl.BlockSpec((S, 128), lambda i: (0, 0)),
        ],
        out_shape=[
            jax.ShapeDtypeStruct((N, D), jnp.float32),
            jax.ShapeDtypeStruct((S, 128), jnp.float32),
        ],
    )(x, seg.reshape(N // T, 1, T), W1, b1.reshape(1, H1), W2,
      b2.reshape(1, H2), W3.reshape(1, H2), b3.reshape(1, 1))


def _seg_body(y_hbm, seg_hbm, pn_hbm, xbuf, segbuf, acc):
    c = lax.axis_index("c")
    s = lax.axis_index("s")
    base = s * RW
    nfull = lax.select(s == 15, (N - 15 * RW) // CH, RW // CH)
    ntail = lax.select(s == 15, (N - 15 * RW - ((N - 15 * RW) // CH) * CH) // 8,
                       0)
    col0 = c * CW
    iota = lax.broadcasted_iota(jnp.int32, (16,), 0)

    def zbody(k, carry):
        acc[k // (CW // 16), pl.ds((k % (CW // 16)) * 16, 16)] = jnp.zeros(
            (16,), jnp.float32)
        return carry

    lax.fori_loop(0, S * CW // 16, zbody, 0)

    def row_block(nrows):
        def rbody(r, carry):
            rvec = jnp.full((16,), r, jnp.int32)
            segsplat = plsc.load_gather(segbuf, [rvec])
            for j in range(CW // 16):
                v = xbuf[r, pl.ds(16 * j, 16)]
                plsc.addupdate_scatter(acc, [segsplat, iota + 16 * j], v)
            return carry

        lax.fori_loop(0, nrows, rbody, 0)

    def body(i, carry):
        off = base + i * CH
        pltpu.sync_copy(y_hbm.at[pl.ds(off, CH), pl.ds(col0, CW)], xbuf)
        pltpu.sync_copy(seg_hbm.at[pl.ds(off, CH)], segbuf)
        row_block(CH)
        return carry

    lax.fori_loop(0, nfull, body, 0)

    def tbody(i, carry):
        off = base + nfull * CH + i * 8
        pltpu.sync_copy(y_hbm.at[pl.ds(off, 8), pl.ds(col0, CW)],
                        xbuf.at[pl.ds(0, 8)])
        pltpu.sync_copy(seg_hbm.at[pl.ds(off, 8)], segbuf.at[pl.ds(0, 8)])
        row_block(8)
        return carry

    lax.fori_loop(0, ntail, tbody, 0)

    pltpu.sync_copy(acc, pn_hbm.at[s, :, pl.ds(col0, CW)])


def _seg_call(y, seg):
    mesh = plsc.VectorSubcoreMesh(core_axis_name="c", subcore_axis_name="s")
    f = pl.kernel(
        _seg_body,
        out_type=jax.ShapeDtypeStruct((16, S, D), jnp.float32),
        mesh=mesh,
        compiler_params=pltpu.CompilerParams(needs_layout_passes=False),
        scratch_types=[
            pltpu.VMEM((CH, CW), jnp.float32),
            pltpu.VMEM((CH,), jnp.int32),
            pltpu.VMEM((S, CW), jnp.float32),
        ],
    )
    return f(y, seg)


def _fin_body(pn_ref, den_ref, o_ref):
    i = pl.program_id(0)

    @pl.when(i == 0)
    def _init():
        o_ref[...] = jnp.zeros_like(o_ref)

    o_ref[...] += pn_ref[0]

    @pl.when(i == 15)
    def _done():
        o_ref[...] = o_ref[...] / den_ref[:, 0:1]


def _fin_call(pn, den):
    return pl.pallas_call(
        _fin_body,
        grid=(16,),
        in_specs=[
            pl.BlockSpec((1, S, D), lambda i: (i, 0, 0)),
            pl.BlockSpec((S, 128), lambda i: (0, 0)),
        ],
        out_specs=pl.BlockSpec((S, D), lambda i: (0, 0)),
        out_shape=jax.ShapeDtypeStruct((S, D), jnp.float32),
    )(pn, den)


def kernel(x, batch_index, W1, b1, W2, b2, W3, b3):
    seg = batch_index.astype(jnp.int32)
    y, den = _mlp_call(x, seg, W1, b1, W2, b2, W3, b3)
    pn = _seg_call(y, seg)
    return _fin_call(pn, den)


# speedup vs baseline: 1.7596x; 1.2082x over previous
import functools

import jax
import jax.numpy as jnp
from jax import lax
from jax.experimental import pallas as pl
from jax.experimental.pallas import tpu as pltpu
from jax.experimental.pallas import tpu_sc as plsc

N = 100000
D = 512
S = 256
H1, H2 = 256, 128

T = 2000

RW = 6272
CH = 128
CW = 256


def _mlp_body(x_ref, seg_ref, w1_ref, b1_ref, w2_ref, b2_ref, w3_ref, b3_ref,
              y_ref, den_ref):
    i = pl.program_id(0)
    x = x_ref[...]
    h = jnp.maximum(
        jnp.dot(x, w1_ref[...], preferred_element_type=jnp.float32)
        + b1_ref[...], 0.0)
    h = jnp.maximum(
        jnp.dot(h, w2_ref[...], preferred_element_type=jnp.float32)
        + b2_ref[...], 0.0)
    logit = jnp.sum(h * w3_ref[...], axis=1, keepdims=True) + b3_ref[...]
    a = jnp.exp(logit)
    y_ref[...] = x * a

    seg = seg_ref[0, 0, :]
    onehot = (seg[None, :] == lax.broadcasted_iota(jnp.int32, (S, T), 0)
              ).astype(jnp.float32)
    a_b = jnp.broadcast_to(a, (a.shape[0], 128))

    @pl.when(i == 0)
    def _init():
        den_ref[...] = jnp.zeros_like(den_ref)

    den_ref[...] += jnp.dot(onehot, a_b, preferred_element_type=jnp.float32)


def _mlp_call(x, seg, W1, b1, W2, b2, W3, b3):
    grid = (N // T,)
    return pl.pallas_call(
        _mlp_body,
        grid=grid,
        in_specs=[
            pl.BlockSpec((T, D), lambda i: (i, 0)),
            pl.BlockSpec((1, 1, T), lambda i: (i, 0, 0)),
            pl.BlockSpec((D, H1), lambda i: (0, 0)),
            pl.BlockSpec((1, H1), lambda i: (0, 0)),
            pl.BlockSpec((H1, H2), lambda i: (0, 0)),
            pl.BlockSpec((1, H2), lambda i: (0, 0)),
            pl.BlockSpec((1, H2), lambda i: (0, 0)),
            pl.BlockSpec((1, 1), lambda i: (0, 0)),
        ],
        out_specs=[
            pl.BlockSpec((T, D), lambda i: (i, 0)),
            pl.BlockSpec((S, 128), lambda i: (0, 0)),
        ],
        out_shape=[
            jax.ShapeDtypeStruct((N, D), jnp.float32),
            jax.ShapeDtypeStruct((S, 128), jnp.float32),
        ],
    )(x, seg.reshape(N // T, 1, T), W1, b1.reshape(1, H1), W2,
      b2.reshape(1, H2), W3.reshape(1, H2), b3.reshape(1, 1))


def _seg_body(y_hbm, seg_hbm, pn_hbm, xbuf, segbuf, acc):
    c = lax.axis_index("c")
    s = lax.axis_index("s")
    base = s * RW
    nfull = lax.select(s == 15, (N - 15 * RW) // CH, RW // CH)
    ntail = lax.select(s == 15, (N - 15 * RW - ((N - 15 * RW) // CH) * CH) // 8,
                       0)
    col0 = c * CW
    iota = lax.broadcasted_iota(jnp.int32, (16,), 0)

    def zbody(k, carry):
        acc[k // (CW // 16), pl.ds((k % (CW // 16)) * 16, 16)] = jnp.zeros(
            (16,), jnp.float32)
        return carry

    lax.fori_loop(0, S * CW // 16, zbody, 0)

    NJ = CW // 16
    zero16 = jnp.zeros((16,), jnp.float32)

    def row_block(nrows):
        def seg_at(r):
            rvec = jnp.full((16,), r, jnp.int32)
            return jnp.max(plsc.load_gather(segbuf, [rvec]))

        def flush(prev, cregs):
            for j in range(NJ):
                plsc.addupdate(acc.at[prev, pl.ds(16 * j, 16)], cregs[j])
            return (zero16,) * NJ

        def rbody(r, carry):
            prev, cregs = carry
            seg_r = seg_at(r)
            cregs = lax.cond(seg_r != prev,
                             lambda cr: flush(prev, cr),
                             lambda cr: cr, cregs)
            cregs = tuple(cregs[j] + xbuf[r, pl.ds(16 * j, 16)]
                          for j in range(NJ))
            return (seg_r, cregs)

        prev0 = seg_at(0)
        prev, cregs = lax.fori_loop(0, nrows, rbody, (prev0, (zero16,) * NJ))
        flush(prev, cregs)

    def body(i, carry):
        off = base + i * CH
        pltpu.sync_copy(y_hbm.at[pl.ds(off, CH), pl.ds(col0, CW)], xbuf)
        pltpu.sync_copy(seg_hbm.at[pl.ds(off, CH)], segbuf)
        row_block(CH)
        return carry

    lax.fori_loop(0, nfull, body, 0)

    def tbody(i, carry):
        off = base + nfull * CH + i * 8
        pltpu.sync_copy(y_hbm.at[pl.ds(off, 8), pl.ds(col0, CW)],
                        xbuf.at[pl.ds(0, 8)])
        pltpu.sync_copy(seg_hbm.at[pl.ds(off, 8)], segbuf.at[pl.ds(0, 8)])
        row_block(8)
        return carry

    lax.fori_loop(0, ntail, tbody, 0)

    pltpu.sync_copy(acc, pn_hbm.at[s, :, pl.ds(col0, CW)])


def _seg_call(y, seg):
    mesh = plsc.VectorSubcoreMesh(core_axis_name="c", subcore_axis_name="s")
    f = pl.kernel(
        _seg_body,
        out_type=jax.ShapeDtypeStruct((16, S, D), jnp.float32),
        mesh=mesh,
        compiler_params=pltpu.CompilerParams(needs_layout_passes=False),
        scratch_types=[
            pltpu.VMEM((CH, CW), jnp.float32),
            pltpu.VMEM((CH,), jnp.int32),
            pltpu.VMEM((S, CW), jnp.float32),
        ],
    )
    return f(y, seg)


def _fin_body(pn_ref, den_ref, o_ref):
    i = pl.program_id(0)

    @pl.when(i == 0)
    def _init():
        o_ref[...] = jnp.zeros_like(o_ref)

    o_ref[...] += pn_ref[0]

    @pl.when(i == 15)
    def _done():
        o_ref[...] = o_ref[...] / den_ref[:, 0:1]


def _fin_call(pn, den):
    return pl.pallas_call(
        _fin_body,
        grid=(16,),
        in_specs=[
            pl.BlockSpec((1, S, D), lambda i: (i, 0, 0)),
            pl.BlockSpec((S, 128), lambda i: (0, 0)),
        ],
        out_specs=pl.BlockSpec((S, D), lambda i: (0, 0)),
        out_shape=jax.ShapeDtypeStruct((S, D), jnp.float32),
    )(pn, den)


def kernel(x, batch_index, W1, b1, W2, b2, W3, b3):
    seg = batch_index.astype(jnp.int32)
    y, den = _mlp_call(x, seg, W1, b1, W2, b2, W3, b3)
    pn = _seg_call(y, seg)
    return _fin_call(pn, den)


# speedup vs baseline: 2.0153x; 1.1453x over previous
import functools

import jax
import jax.numpy as jnp
from jax import lax
from jax.experimental import pallas as pl
from jax.experimental.pallas import tpu as pltpu
from jax.experimental.pallas import tpu_sc as plsc

N = 100000
D = 512
S = 256
H1, H2 = 256, 128

T = 2000

RW = 6272
CH = 128
CW = 256


def _mlp_body(x_ref, seg_ref, w1_ref, b1_ref, w2_ref, b2_ref, w3_ref, b3_ref,
              y_ref, den_ref):
    i = pl.program_id(0)
    x = x_ref[...]
    h = jnp.maximum(
        jnp.dot(x, w1_ref[...], preferred_element_type=jnp.float32)
        + b1_ref[...], 0.0)
    h = jnp.maximum(
        jnp.dot(h, w2_ref[...], preferred_element_type=jnp.float32)
        + b2_ref[...], 0.0)
    logit = jnp.sum(h * w3_ref[...], axis=1, keepdims=True) + b3_ref[...]
    a = jnp.exp(logit)
    y_ref[...] = x * a

    seg = seg_ref[0, 0, :]
    onehot = (seg[None, :] == lax.broadcasted_iota(jnp.int32, (S, T), 0)
              ).astype(jnp.float32)
    a_b = jnp.broadcast_to(a, (a.shape[0], 128))

    @pl.when(i == 0)
    def _init():
        den_ref[...] = jnp.zeros_like(den_ref)

    den_ref[...] += jnp.dot(onehot, a_b, preferred_element_type=jnp.float32)


def _mlp_call(x, seg, W1, b1, W2, b2, W3, b3):
    grid = (N // T,)
    return pl.pallas_call(
        _mlp_body,
        grid=grid,
        in_specs=[
            pl.BlockSpec((T, D), lambda i: (i, 0)),
            pl.BlockSpec((1, 1, T), lambda i: (i, 0, 0)),
            pl.BlockSpec((D, H1), lambda i: (0, 0)),
            pl.BlockSpec((1, H1), lambda i: (0, 0)),
            pl.BlockSpec((H1, H2), lambda i: (0, 0)),
            pl.BlockSpec((1, H2), lambda i: (0, 0)),
            pl.BlockSpec((1, H2), lambda i: (0, 0)),
            pl.BlockSpec((1, 1), lambda i: (0, 0)),
        ],
        out_specs=[
            pl.BlockSpec((T, D), lambda i: (i, 0)),
            pl.BlockSpec((S, 128), lambda i: (0, 0)),
        ],
        out_shape=[
            jax.ShapeDtypeStruct((N, D), jnp.float32),
            jax.ShapeDtypeStruct((S, 128), jnp.float32),
        ],
    )(x, seg.reshape(N // T, 1, T), W1, b1.reshape(1, H1), W2,
      b2.reshape(1, H2), W3.reshape(1, H2), b3.reshape(1, 1))


def _seg_body(y_hbm, seg_hbm, pn_hbm, xbuf, segbuf, acc):
    c = lax.axis_index("c")
    s = lax.axis_index("s")
    base = s * RW
    nfull = lax.select(s == 15, (N - 15 * RW) // CH, RW // CH)
    ntail = lax.select(s == 15, (N - 15 * RW - ((N - 15 * RW) // CH) * CH) // 8,
                       0)
    col0 = c * CW
    iota = lax.broadcasted_iota(jnp.int32, (16,), 0)

    def zbody(k, carry):
        acc[k // (CW // 16), pl.ds((k % (CW // 16)) * 16, 16)] = jnp.zeros(
            (16,), jnp.float32)
        return carry

    lax.fori_loop(0, S * CW // 16, zbody, 0)

    NJ = CW // 16
    zero16 = jnp.zeros((16,), jnp.float32)

    def row_block(nrows):
        def seg_at(r):
            rvec = jnp.full((16,), r, jnp.int32)
            return jnp.max(plsc.load_gather(segbuf, [rvec]))

        def flush(prev, cregs):
            for j in range(NJ):
                plsc.addupdate(acc.at[prev, pl.ds(16 * j, 16)], cregs[j])
            return (zero16,) * NJ

        def rbody(r, carry):
            prev, cregs = carry
            seg_r = seg_at(r)
            cregs = lax.cond(seg_r != prev,
                             lambda cr: flush(prev, cr),
                             lambda cr: cr, cregs)
            cregs = tuple(cregs[j] + xbuf[r, pl.ds(16 * j, 16)]
                          for j in range(NJ))
            return (seg_r, cregs)

        def gbody(gi, carry):
            prev, cregs = carry
            r0 = gi * 16
            va = segbuf[pl.ds(r0, 16)]
            uniform = (jnp.max(va) == prev) & (jnp.min(va) == prev)

            def fast(carry_in):
                _, cr = carry_in
                for rr in range(16):
                    cr = tuple(cr[j] + xbuf[r0 + rr, pl.ds(16 * j, 16)]
                               for j in range(NJ))
                return (prev, cr)

            def slow(carry_in):
                return lax.fori_loop(r0, r0 + 16, rbody, carry_in)

            return lax.cond(uniform, fast, slow, (prev, cregs))

        prev0 = seg_at(0)
        init = (prev0, (zero16,) * NJ)
        if nrows % 16 == 0:
            prev, cregs = lax.fori_loop(0, nrows // 16, gbody, init)
        else:
            prev, cregs = lax.fori_loop(0, nrows, rbody, init)
        flush(prev, cregs)

    def body(i, carry):
        off = base + i * CH
        pltpu.sync_copy(y_hbm.at[pl.ds(off, CH), pl.ds(col0, CW)], xbuf)
        pltpu.sync_copy(seg_hbm.at[pl.ds(off, CH)], segbuf)
        row_block(CH)
        return carry

    lax.fori_loop(0, nfull, body, 0)

    def tbody(i, carry):
        off = base + nfull * CH + i * 8
        pltpu.sync_copy(y_hbm.at[pl.ds(off, 8), pl.ds(col0, CW)],
                        xbuf.at[pl.ds(0, 8)])
        pltpu.sync_copy(seg_hbm.at[pl.ds(off, 8)], segbuf.at[pl.ds(0, 8)])
        row_block(8)
        return carry

    lax.fori_loop(0, ntail, tbody, 0)

    pltpu.sync_copy(acc, pn_hbm.at[s, :, pl.ds(col0, CW)])


def _seg_call(y, seg):
    mesh = plsc.VectorSubcoreMesh(core_axis_name="c", subcore_axis_name="s")
    f = pl.kernel(
        _seg_body,
        out_type=jax.ShapeDtypeStruct((16, S, D), jnp.float32),
        mesh=mesh,
        compiler_params=pltpu.CompilerParams(needs_layout_passes=False),
        scratch_types=[
            pltpu.VMEM((CH, CW), jnp.float32),
            pltpu.VMEM((CH,), jnp.int32),
            pltpu.VMEM((S, CW), jnp.float32),
        ],
    )
    return f(y, seg)


def _fin_body(pn_ref, den_ref, o_ref):
    i = pl.program_id(0)

    @pl.when(i == 0)
    def _init():
        o_ref[...] = jnp.zeros_like(o_ref)

    o_ref[...] += pn_ref[0]

    @pl.when(i == 15)
    def _done():
        o_ref[...] = o_ref[...] / den_ref[:, 0:1]


def _fin_call(pn, den):
    return pl.pallas_call(
        _fin_body,
        grid=(16,),
        in_specs=[
            pl.BlockSpec((1, S, D), lambda i: (i, 0, 0)),
            pl.BlockSpec((S, 128), lambda i: (0, 0)),
        ],
        out_specs=pl.BlockSpec((S, D), lambda i: (0, 0)),
        out_shape=jax.ShapeDtypeStruct((S, D), jnp.float32),
    )(pn, den)


def kernel(x, batch_index, W1, b1, W2, b2, W3, b3):
    seg = batch_index.astype(jnp.int32)
    y, den = _mlp_call(x, seg, W1, b1, W2, b2, W3, b3)
    pn = _seg_call(y, seg)
    return _fin_call(pn, den)
